# Initial kernel scaffold; baseline (speedup 1.0000x reference)
import jax
import jax.numpy as jnp
from jax.experimental import pallas as pl


def kernel(x, W, a_src, a_dst, bias):
    raise NotImplementedError("write your pallas kernel here")


# speedup vs baseline: 1.8674x; 1.8674x over previous
import jax
import jax.numpy as jnp
from jax.experimental import pallas as pl
from jax.experimental.pallas import tpu as pltpu

_B, _N, _D, _O = 32, 512, 128, 128


def _fused_attention_kernel(x_ref, W_ref, asrc_ref, adst_ref, bias_ref, o_ref):
    x = x_ref[0]
    W = W_ref[...]
    H = jnp.dot(x, W, preferred_element_type=jnp.float32)

    a_src = asrc_ref[...]
    a_dst = adst_ref[...]
    d_col = jax.lax.dot_general(H, a_dst, (((1,), (1,)), ((), ())),
                                preferred_element_type=jnp.float32)
    s_row = jax.lax.dot_general(a_src, H, (((1,), (1,)), ((), ())),
                                preferred_element_type=jnp.float32)

    e = d_col + s_row
    e = jnp.where(e >= 0, e, 0.2 * e)
    m = jnp.max(e, axis=1, keepdims=True)
    p = jnp.exp(e - m)
    attn = p / jnp.sum(p, axis=1, keepdims=True)

    out = jnp.dot(attn, H, preferred_element_type=jnp.float32)
    out = out + bias_ref[...]
    o_ref[0] = jnp.where(out > 0, out, jnp.exp(out) - 1.0)


def kernel(x, W, a_src, a_dst, bias):
    grid = (_B,)
    out = pl.pallas_call(
        _fused_attention_kernel,
        grid=grid,
        in_specs=[
            pl.BlockSpec((1, _N, _D), lambda b: (b, 0, 0)),
            pl.BlockSpec((_D, _O), lambda b: (0, 0)),
            pl.BlockSpec((1, _O), lambda b: (0, 0)),
            pl.BlockSpec((1, _O), lambda b: (0, 0)),
            pl.BlockSpec((1, _O), lambda b: (0, 0)),
        ],
        out_specs=pl.BlockSpec((1, _N, _O), lambda b: (b, 0, 0)),
        out_shape=jax.ShapeDtypeStruct((_B, _N, _O), jnp.float32),
    )(x, W, a_src.reshape(1, _O), a_dst.reshape(1, _O), bias.reshape(1, _O))
    return out
